# Initial kernel scaffold; baseline (speedup 1.0000x reference)
import jax
import jax.numpy as jnp
from jax.experimental import pallas as pl


def kernel(x, Wq, bq, Wv, bv, hyperplanes):
    raise NotImplementedError("write your pallas kernel here")


# speedup vs baseline: 1.7084x; 1.7084x over previous
import math

import jax
import jax.numpy as jnp
from jax.experimental import pallas as pl

_EMBED = 768
_HEADS = 12
_DH = _EMBED // _HEADS
_NB = 4
_ROW_TILE = 512


def _lsh_attn_kernel(x_ref, wq_ref, bq_ref, wv_ref, bv_ref, hyp_ref, out_ref):
    x = x_ref[...]
    wq = wq_ref[0]
    wv = wv_ref[0]
    bq = bq_ref[0]
    bv = bv_ref[0]
    hyp = hyp_ref[...]

    n = x.shape[0]
    q = jax.lax.dot_general(
        x, wq, (((1,), (1,)), ((), ())), preferred_element_type=jnp.float32
    ) + bq
    v = jax.lax.dot_general(
        x, wv, (((1,), (1,)), ((), ())), preferred_element_type=jnp.float32
    ) + bv

    proj = jnp.dot(q, hyp[:_DH, :], preferred_element_type=jnp.float32)
    proj = proj + hyp[_DH:_DH + 1, :]
    bits = (proj >= 0).astype(jnp.int32)
    bucket = bits[:, 0:1] + 2 * bits[:, 1:2]
    iota = jax.lax.broadcasted_iota(jnp.int32, (n, _NB), 1)
    onehot = (bucket == iota).astype(jnp.float32)

    scale = 1.0 / math.sqrt(_EMBED)
    for t in range(n // _ROW_TILE):
        sl = slice(t * _ROW_TILE, (t + 1) * _ROW_TILE)
        qt = q[sl, :]
        oht = onehot[sl, :]
        s = jax.lax.dot_general(
            qt, q, (((1,), (1,)), ((), ())), preferred_element_type=jnp.float32
        ) * scale
        mask = jax.lax.dot_general(
            oht, onehot, (((1,), (1,)), ((), ())),
            preferred_element_type=jnp.float32,
        )
        s = s * mask
        m = jnp.max(s, axis=-1, keepdims=True)
        p = jnp.exp(s - m)
        p = p / jnp.sum(p, axis=-1, keepdims=True)
        out_ref[0, sl, :] = jnp.dot(p, v, preferred_element_type=jnp.float32)


def kernel(x, Wq, bq, Wv, bv, hyperplanes):
    b, n, e = x.shape
    h = _HEADS
    dh = e // h
    x2 = x[0]
    wq3 = Wq.reshape(h, dh, e)
    wv3 = Wv.reshape(h, dh, e)
    bq3 = bq.reshape(h, 1, dh)
    bv3 = bv.reshape(h, 1, dh)

    out = pl.pallas_call(
        _lsh_attn_kernel,
        grid=(h,),
        in_specs=[
            pl.BlockSpec((n, e), lambda i: (0, 0)),
            pl.BlockSpec((1, dh, e), lambda i: (i, 0, 0)),
            pl.BlockSpec((1, 1, dh), lambda i: (i, 0, 0)),
            pl.BlockSpec((1, dh, e), lambda i: (i, 0, 0)),
            pl.BlockSpec((1, 1, dh), lambda i: (i, 0, 0)),
            pl.BlockSpec((dh + 1, 2), lambda i: (0, 0)),
        ],
        out_specs=pl.BlockSpec((1, n, dh), lambda i: (i, 0, 0)),
        out_shape=jax.ShapeDtypeStruct((h, n, dh), jnp.float32),
    )(x2, wq3, bq3, wv3, bv3, hyperplanes)

    return out.transpose(1, 0, 2).reshape(b, n, e)


# speedup vs baseline: 1.7314x; 1.0135x over previous
import math

import jax
import jax.numpy as jnp
from jax.experimental import pallas as pl

_EMBED = 768
_HEADS = 12
_DH = _EMBED // _HEADS
_NB = 4
_ROW_TILE = 512


def _lsh_attn_kernel(x_ref, wq_ref, bq_ref, wv_ref, bv_ref, hyp_ref, out_ref):
    x = x_ref[...]
    wq = wq_ref[0]
    wv = wv_ref[0]
    bq = bq_ref[0]
    bv = bv_ref[0]
    hyp = hyp_ref[...]

    n = x.shape[0]
    q = jax.lax.dot_general(
        x, wq, (((1,), (1,)), ((), ())), preferred_element_type=jnp.float32
    ) + bq
    v = jax.lax.dot_general(
        x, wv, (((1,), (1,)), ((), ())), preferred_element_type=jnp.float32
    ) + bv

    proj = jnp.dot(q, hyp[:_DH, :], preferred_element_type=jnp.float32)
    proj = proj + hyp[_DH:_DH + 1, :]
    bits = (proj >= 0).astype(jnp.int32)
    bucket = bits[:, 0:1] + 2 * bits[:, 1:2]
    iota = jax.lax.broadcasted_iota(jnp.int32, (n, _NB), 1)
    onehot = (bucket == iota).astype(jnp.float32)

    scale = 1.0 / math.sqrt(_EMBED)
    qs = (q * scale).astype(jnp.bfloat16)
    qk = q.astype(jnp.bfloat16)
    ohb = onehot.astype(jnp.bfloat16)
    for t in range(n // _ROW_TILE):
        sl = slice(t * _ROW_TILE, (t + 1) * _ROW_TILE)
        s = jax.lax.dot_general(
            qs[sl, :], qk, (((1,), (1,)), ((), ())),
            preferred_element_type=jnp.float32,
        )
        mask = jax.lax.dot_general(
            ohb[sl, :], ohb, (((1,), (1,)), ((), ())),
            preferred_element_type=jnp.float32,
        )
        s = s * mask
        m = jnp.max(s, axis=-1, keepdims=True)
        p = jnp.exp(s - m)
        p = p / jnp.sum(p, axis=-1, keepdims=True)
        out_ref[0, sl, :] = jnp.dot(p, v, preferred_element_type=jnp.float32)


def kernel(x, Wq, bq, Wv, bv, hyperplanes):
    b, n, e = x.shape
    h = _HEADS
    dh = e // h
    x2 = x[0]
    wq3 = Wq.reshape(h, dh, e)
    wv3 = Wv.reshape(h, dh, e)
    bq3 = bq.reshape(h, 1, dh)
    bv3 = bv.reshape(h, 1, dh)

    out = pl.pallas_call(
        _lsh_attn_kernel,
        grid=(h,),
        in_specs=[
            pl.BlockSpec((n, e), lambda i: (0, 0)),
            pl.BlockSpec((1, dh, e), lambda i: (i, 0, 0)),
            pl.BlockSpec((1, 1, dh), lambda i: (i, 0, 0)),
            pl.BlockSpec((1, dh, e), lambda i: (i, 0, 0)),
            pl.BlockSpec((1, 1, dh), lambda i: (i, 0, 0)),
            pl.BlockSpec((dh + 1, 2), lambda i: (0, 0)),
        ],
        out_specs=pl.BlockSpec((1, n, dh), lambda i: (i, 0, 0)),
        out_shape=jax.ShapeDtypeStruct((h, n, dh), jnp.float32),
    )(x2, wq3, bq3, wv3, bv3, hyperplanes)

    return out.transpose(1, 0, 2).reshape(b, n, e)


# speedup vs baseline: 2.1644x; 1.2501x over previous
import math

import jax
import jax.numpy as jnp
from jax.experimental import pallas as pl

_EMBED = 768
_HEADS = 12
_DH = _EMBED // _HEADS
_NB = 4
_ROW_TILE = 512


def _lsh_attn_kernel(x_ref, wq_ref, bq_ref, wv_ref, bv_ref, hyp_ref, out_ref):
    x = x_ref[...]
    wq = wq_ref[0]
    wv = wv_ref[0]
    bq = bq_ref[0]
    bv = bv_ref[0]
    hyp = hyp_ref[...]

    n = x.shape[0]
    q = jax.lax.dot_general(
        x, wq, (((1,), (1,)), ((), ())), preferred_element_type=jnp.float32
    ) + bq
    v = jax.lax.dot_general(
        x, wv, (((1,), (1,)), ((), ())), preferred_element_type=jnp.float32
    ) + bv

    proj = jnp.dot(q, hyp[:_DH, :], preferred_element_type=jnp.float32)
    proj = proj + hyp[_DH:_DH + 1, :]
    bits = (proj >= 0).astype(jnp.int32)
    bucket = bits[:, 0:1] + 2 * bits[:, 1:2]
    iota = jax.lax.broadcasted_iota(jnp.int32, (n, _NB), 1)
    onehot = (bucket == iota).astype(jnp.float32)

    scale = 1.0 / math.sqrt(_EMBED)
    qs = (q * scale).astype(jnp.bfloat16)
    qk = q.astype(jnp.bfloat16)
    ohb = onehot.astype(jnp.bfloat16)
    vext = jnp.concatenate(
        [v, jnp.ones((n, 1), dtype=jnp.float32)], axis=1
    )
    for t in range(n // _ROW_TILE):
        sl = slice(t * _ROW_TILE, (t + 1) * _ROW_TILE)
        s = jax.lax.dot_general(
            qs[sl, :], qk, (((1,), (1,)), ((), ())),
            preferred_element_type=jnp.float32,
        )
        mask = jax.lax.dot_general(
            ohb[sl, :], ohb, (((1,), (1,)), ((), ())),
            preferred_element_type=jnp.float32,
        )
        p = jnp.exp(s * mask)
        av = jnp.dot(p, vext, preferred_element_type=jnp.float32)
        out_ref[0, sl, :] = av[:, :_DH] * (1.0 / av[:, _DH:_DH + 1])


def kernel(x, Wq, bq, Wv, bv, hyperplanes):
    b, n, e = x.shape
    h = _HEADS
    dh = e // h
    x2 = x[0]
    wq3 = Wq.reshape(h, dh, e)
    wv3 = Wv.reshape(h, dh, e)
    bq3 = bq.reshape(h, 1, dh)
    bv3 = bv.reshape(h, 1, dh)

    out = pl.pallas_call(
        _lsh_attn_kernel,
        grid=(h,),
        in_specs=[
            pl.BlockSpec((n, e), lambda i: (0, 0)),
            pl.BlockSpec((1, dh, e), lambda i: (i, 0, 0)),
            pl.BlockSpec((1, 1, dh), lambda i: (i, 0, 0)),
            pl.BlockSpec((1, dh, e), lambda i: (i, 0, 0)),
            pl.BlockSpec((1, 1, dh), lambda i: (i, 0, 0)),
            pl.BlockSpec((dh + 1, 2), lambda i: (0, 0)),
        ],
        out_specs=pl.BlockSpec((1, n, dh), lambda i: (i, 0, 0)),
        out_shape=jax.ShapeDtypeStruct((h, n, dh), jnp.float32),
    )(x2, wq3, bq3, wv3, bv3, hyperplanes)

    return out.transpose(1, 0, 2).reshape(b, n, e)


# speedup vs baseline: 2.1680x; 1.0017x over previous
import math

import jax
import jax.numpy as jnp
from jax.experimental import pallas as pl

_EMBED = 768
_HEADS = 12
_DH = _EMBED // _HEADS
_NB = 4
_ROW_TILE = 512


def _lsh_attn_kernel(x_ref, wq_ref, bq_ref, wv_ref, bv_ref, hyp_ref, out_ref):
    x = x_ref[...]
    wq = wq_ref[0]
    wv = wv_ref[0]
    bq = bq_ref[0]
    bv = bv_ref[0]
    hyp = hyp_ref[...]

    n = x.shape[0]
    q = jax.lax.dot_general(
        x, wq, (((1,), (1,)), ((), ())), preferred_element_type=jnp.float32
    ) + bq
    v = jax.lax.dot_general(
        x.astype(jnp.bfloat16), wv.astype(jnp.bfloat16),
        (((1,), (1,)), ((), ())), preferred_element_type=jnp.float32
    ) + bv

    proj = jnp.dot(q, hyp[:_DH, :], preferred_element_type=jnp.float32)
    proj = proj + hyp[_DH:_DH + 1, :]
    bits = (proj >= 0).astype(jnp.int32)
    bucket = bits[:, 0:1] + 2 * bits[:, 1:2]
    iota = jax.lax.broadcasted_iota(jnp.int32, (n, _NB), 1)
    onehot = (bucket == iota).astype(jnp.float32)

    scale = 1.0 / math.sqrt(_EMBED)
    qs = (q * scale).astype(jnp.bfloat16)
    qk = q.astype(jnp.bfloat16)
    ohb = onehot.astype(jnp.bfloat16)
    vext = jnp.concatenate(
        [v, jnp.ones((n, 1), dtype=jnp.float32)], axis=1
    ).astype(jnp.bfloat16)
    for t in range(n // _ROW_TILE):
        sl = slice(t * _ROW_TILE, (t + 1) * _ROW_TILE)
        s = jax.lax.dot_general(
            qs[sl, :], qk, (((1,), (1,)), ((), ())),
            preferred_element_type=jnp.float32,
        )
        mask = jax.lax.dot_general(
            ohb[sl, :], ohb, (((1,), (1,)), ((), ())),
            preferred_element_type=jnp.float32,
        )
        p = jnp.exp(s * mask).astype(jnp.bfloat16)
        av = jnp.dot(p, vext, preferred_element_type=jnp.float32)
        out_ref[0, sl, :] = av[:, :_DH] * (1.0 / av[:, _DH:_DH + 1])


def kernel(x, Wq, bq, Wv, bv, hyperplanes):
    b, n, e = x.shape
    h = _HEADS
    dh = e // h
    x2 = x[0]
    wq3 = Wq.reshape(h, dh, e)
    wv3 = Wv.reshape(h, dh, e)
    bq3 = bq.reshape(h, 1, dh)
    bv3 = bv.reshape(h, 1, dh)

    out = pl.pallas_call(
        _lsh_attn_kernel,
        grid=(h,),
        in_specs=[
            pl.BlockSpec((n, e), lambda i: (0, 0)),
            pl.BlockSpec((1, dh, e), lambda i: (i, 0, 0)),
            pl.BlockSpec((1, 1, dh), lambda i: (i, 0, 0)),
            pl.BlockSpec((1, dh, e), lambda i: (i, 0, 0)),
            pl.BlockSpec((1, 1, dh), lambda i: (i, 0, 0)),
            pl.BlockSpec((dh + 1, 2), lambda i: (0, 0)),
        ],
        out_specs=pl.BlockSpec((1, n, dh), lambda i: (i, 0, 0)),
        out_shape=jax.ShapeDtypeStruct((h, n, dh), jnp.float32),
    )(x2, wq3, bq3, wv3, bv3, hyperplanes)

    return out.transpose(1, 0, 2).reshape(b, n, e)


# speedup vs baseline: 3.1054x; 1.4324x over previous
import math

import jax
import jax.numpy as jnp
from jax.experimental import pallas as pl

_EMBED = 768
_HEADS = 12
_DH = _EMBED // _HEADS
_NB = 4
_ROW_TILE = 512
_HPP = 2


def _lsh_attn_kernel(x_ref, wq_ref, bq_ref, wv_ref, bv_ref, hyp_ref, out_ref):
    x = x_ref[...]
    wq = wq_ref[0]
    wv = wv_ref[0]
    bq = bq_ref[0]
    bv = bv_ref[0]
    hyp = hyp_ref[...]

    n = x.shape[0]
    qp = jax.lax.dot_general(
        x, wq, (((1,), (1,)), ((), ())), preferred_element_type=jnp.float32
    ) + bq
    vp = jax.lax.dot_general(
        x.astype(jnp.bfloat16), wv.astype(jnp.bfloat16),
        (((1,), (1,)), ((), ())), preferred_element_type=jnp.float32
    ) + bv

    scale = 1.0 / math.sqrt(_EMBED)
    iota = jax.lax.broadcasted_iota(jnp.int32, (n, _NB), 1)
    ones_col = jnp.ones((n, 1), dtype=jnp.float32)

    for h in range(_HPP):
        cs = slice(h * _DH, (h + 1) * _DH)
        q = qp[:, cs]
        v = vp[:, cs]

        proj = jnp.dot(q, hyp[:_DH, :], preferred_element_type=jnp.float32)
        proj = proj + hyp[_DH:_DH + 1, :]
        bits = (proj >= 0).astype(jnp.int32)
        bucket = bits[:, 0:1] + 2 * bits[:, 1:2]
        onehot = (bucket == iota).astype(jnp.bfloat16)

        qs = (q * scale).astype(jnp.bfloat16)
        qk = q.astype(jnp.bfloat16)
        vext = jnp.concatenate([v, ones_col], axis=1).astype(jnp.bfloat16)
        for t in range(n // _ROW_TILE):
            sl = slice(t * _ROW_TILE, (t + 1) * _ROW_TILE)
            s = jax.lax.dot_general(
                qs[sl, :], qk, (((1,), (1,)), ((), ())),
                preferred_element_type=jnp.float32,
            )
            mask = jax.lax.dot_general(
                onehot[sl, :], onehot, (((1,), (1,)), ((), ())),
                preferred_element_type=jnp.float32,
            )
            p = jnp.exp(s * mask).astype(jnp.bfloat16)
            av = jnp.dot(p, vext, preferred_element_type=jnp.float32)
            out_ref[sl, cs] = av[:, :_DH] * (1.0 / av[:, _DH:_DH + 1])


def kernel(x, Wq, bq, Wv, bv, hyperplanes):
    b, n, e = x.shape
    npair = _HEADS // _HPP
    wide = _HPP * _DH
    x2 = x[0]
    wq3 = Wq.reshape(npair, wide, e)
    wv3 = Wv.reshape(npair, wide, e)
    bq3 = bq.reshape(npair, 1, wide)
    bv3 = bv.reshape(npair, 1, wide)

    out = pl.pallas_call(
        _lsh_attn_kernel,
        grid=(npair,),
        in_specs=[
            pl.BlockSpec((n, e), lambda i: (0, 0)),
            pl.BlockSpec((1, wide, e), lambda i: (i, 0, 0)),
            pl.BlockSpec((1, 1, wide), lambda i: (i, 0, 0)),
            pl.BlockSpec((1, wide, e), lambda i: (i, 0, 0)),
            pl.BlockSpec((1, 1, wide), lambda i: (i, 0, 0)),
            pl.BlockSpec((_DH + 1, 2), lambda i: (0, 0)),
        ],
        out_specs=pl.BlockSpec((n, wide), lambda i: (0, i)),
        out_shape=jax.ShapeDtypeStruct((n, e), jnp.float32),
    )(x2, wq3, bq3, wv3, bv3, hyperplanes)

    return out.reshape(b, n, e)


# speedup vs baseline: 3.1210x; 1.0050x over previous
import math

import jax
import jax.numpy as jnp
from jax.experimental import pallas as pl

_EMBED = 768
_HEADS = 12
_DH = _EMBED // _HEADS
_NB = 4
_ROW_TILE = 512
_HPP = 2


def _lsh_attn_kernel(x_ref, wq_ref, bq_ref, wv_ref, bv_ref, hyp_ref, out_ref):
    x = x_ref[...]
    wq = wq_ref[0]
    wv = wv_ref[0]
    bq = bq_ref[0]
    bv = bv_ref[0]
    hyp = hyp_ref[...]

    n = x.shape[0]
    qp = jax.lax.dot_general(
        x, wq, (((1,), (1,)), ((), ())), preferred_element_type=jnp.float32
    ) + bq
    vp = jax.lax.dot_general(
        x.astype(jnp.bfloat16), wv.astype(jnp.bfloat16),
        (((1,), (1,)), ((), ())), preferred_element_type=jnp.float32
    ) + bv

    scale = math.log2(math.e) / math.sqrt(_EMBED)
    iota = jax.lax.broadcasted_iota(jnp.int32, (n, _NB), 1)
    ones_col = jnp.ones((n, 1), dtype=jnp.float32)

    for h in range(_HPP):
        cs = slice(h * _DH, (h + 1) * _DH)
        q = qp[:, cs]
        v = vp[:, cs]

        proj = jnp.dot(q, hyp[:_DH, :], preferred_element_type=jnp.float32)
        proj = proj + hyp[_DH:_DH + 1, :]
        bits = (proj >= 0).astype(jnp.int32)
        bucket = bits[:, 0:1] + 2 * bits[:, 1:2]
        onehot = (bucket == iota).astype(jnp.bfloat16)

        qs = (q * scale).astype(jnp.bfloat16)
        qk = q.astype(jnp.bfloat16)
        vext = jnp.concatenate([v, ones_col], axis=1).astype(jnp.bfloat16)
        for t in range(n // _ROW_TILE):
            sl = slice(t * _ROW_TILE, (t + 1) * _ROW_TILE)
            s = jax.lax.dot_general(
                qs[sl, :], qk, (((1,), (1,)), ((), ())),
                preferred_element_type=jnp.float32,
            )
            mask = jax.lax.dot_general(
                onehot[sl, :], onehot, (((1,), (1,)), ((), ())),
                preferred_element_type=jnp.float32,
            )
            p = jnp.exp2(s * mask).astype(jnp.bfloat16)
            av = jnp.dot(p, vext, preferred_element_type=jnp.float32)
            out_ref[sl, cs] = av[:, :_DH] * (1.0 / av[:, _DH:_DH + 1])


def kernel(x, Wq, bq, Wv, bv, hyperplanes):
    b, n, e = x.shape
    npair = _HEADS // _HPP
    wide = _HPP * _DH
    x2 = x[0]
    wq3 = Wq.reshape(npair, wide, e)
    wv3 = Wv.reshape(npair, wide, e)
    bq3 = bq.reshape(npair, 1, wide)
    bv3 = bv.reshape(npair, 1, wide)

    out = pl.pallas_call(
        _lsh_attn_kernel,
        grid=(npair,),
        in_specs=[
            pl.BlockSpec((n, e), lambda i: (0, 0)),
            pl.BlockSpec((1, wide, e), lambda i: (i, 0, 0)),
            pl.BlockSpec((1, 1, wide), lambda i: (i, 0, 0)),
            pl.BlockSpec((1, wide, e), lambda i: (i, 0, 0)),
            pl.BlockSpec((1, 1, wide), lambda i: (i, 0, 0)),
            pl.BlockSpec((_DH + 1, 2), lambda i: (0, 0)),
        ],
        out_specs=pl.BlockSpec((n, wide), lambda i: (0, i)),
        out_shape=jax.ShapeDtypeStruct((n, e), jnp.float32),
    )(x2, wq3, bq3, wv3, bv3, hyperplanes)

    return out.reshape(b, n, e)
